# baseline (speedup 1.0000x reference)
import functools

import jax
import jax.numpy as jnp
from jax import lax
from jax.experimental import pallas as pl
from jax.experimental.pallas import tpu as pltpu
from jax.experimental.pallas import tpu_sc as plsc

_HID = 4096
_NE = 64
_NG = 8
_GS = _NE // _NG
_TKG = 4
_SCALE = 2.5
_NEG = -3.0e38
_CW = 128
_WFIX = 262144.0


def _route_chunk(scores, bias, tb):
    sfc = scores + bias

    grows = []
    for g in range(_NG):
        grp = sfc[g * _GS:(g + 1) * _GS, :]
        m1 = jnp.max(grp, axis=0, keepdims=True)
        is_m1 = grp == m1
        nmax = jnp.sum(jnp.where(is_m1, 1.0, 0.0), axis=0, keepdims=True)
        m2 = jnp.max(jnp.where(is_m1, _NEG, grp), axis=0, keepdims=True)
        m2 = jnp.where(nmax > 1.0, m1, m2)
        grows.append(m1 + m2)
    gscores = jnp.concatenate(grows, axis=0)

    giota = jax.lax.broadcasted_iota(jnp.int32, (_NG, tb), 0)
    grank = jnp.zeros((_NG, tb), jnp.float32)
    for g in range(_NG):
        vg = gscores[g:g + 1, :]
        cond = (vg > gscores) | ((vg == gscores) & (giota > g))
        grank = grank + jnp.where(cond, 1.0, 0.0)
    keep = jnp.where(grank < float(_TKG), 1.0, 0.0)
    keep_full = jnp.concatenate(
        [jnp.broadcast_to(keep[g:g + 1, :], (_GS, tb)) for g in range(_NG)],
        axis=0,
    )
    masked = jnp.where(keep_full > 0.5, sfc, 0.0)

    biota = jax.lax.broadcasted_iota(jnp.int32, (_GS, tb), 0)
    mblk = [masked[b * 8:(b + 1) * 8, :] for b in range(8)]
    rblk = [jnp.zeros((8, tb), jnp.float32) for _ in range(8)]
    for j in range(_NE):
        vj = masked[j:j + 1, :]
        jb = j // 8
        for b in range(8):
            if b < jb:
                cond = vj > mblk[b]
            elif b > jb:
                cond = vj >= mblk[b]
            else:
                cond = (vj > mblk[b]) | ((vj == mblk[b]) & (biota > (j - 8 * jb)))
            rblk[b] = rblk[b] + jnp.where(cond, 1.0, 0.0)
    rank = jnp.concatenate(rblk, axis=0)

    denom = jnp.sum(scores, axis=0, keepdims=True) + 1e-20
    wfix = (scores * (_SCALE * _WFIX) / denom).astype(jnp.int32)
    eiota = jax.lax.broadcasted_iota(jnp.int32, (_NE, tb), 0)
    pk = (rank.astype(jnp.int32) << 26) | (eiota << 20) | wfix
    return pk


def _tc_kernel(hs_ref, w_ref, b_ref, pk_ref):
    w = w_ref[...]
    bias = b_ref[...]
    tb = hs_ref.shape[0]
    logits = jax.lax.dot_general(
        w, hs_ref[...], (((1,), (1,)), ((), ())),
        preferred_element_type=jnp.float32,
    )
    for c in range(tb // _CW):
        lo, hi = c * _CW, (c + 1) * _CW
        scores = jax.nn.sigmoid(logits[:, lo:hi])
        pk_ref[:, lo:hi] = _route_chunk(scores, bias, _CW)


def _tc_run(hs, w, b2d, tb, interpret=False):
    n = hs.shape[0]
    return pl.pallas_call(
        _tc_kernel,
        grid=(n // tb,),
        in_specs=[
            pl.BlockSpec((tb, _HID), lambda i: (i, 0)),
            pl.BlockSpec((_NE, _HID), lambda i: (0, 0)),
            pl.BlockSpec((_NE, 1), lambda i: (0, 0)),
        ],
        out_specs=pl.BlockSpec((_NE, tb), lambda i: (0, i)),
        out_shape=jax.ShapeDtypeStruct((_NE, n), jnp.int32),
        interpret=interpret,
    )(hs, w, b2d)


def _sc_scatter(pk):
    n = pk.shape[1]
    info = plsc.get_sparse_core_info()
    nw = info.num_cores * info.num_subcores
    t_per_w = n // nw
    mesh = plsc.VectorSubcoreMesh(core_axis_name="c", subcore_axis_name="s")

    @functools.partial(
        pl.kernel, mesh=mesh,
        out_type=jax.ShapeDtypeStruct((n * _NE,), jnp.int32),
        scratch_types=[
            pltpu.VMEM((_NE, t_per_w), jnp.int32),
            pltpu.VMEM((t_per_w * _NE,), jnp.int32),
        ],
        compiler_params=pltpu.CompilerParams(needs_layout_passes=False),
    )
    def k(pk_hbm, out_hbm, pk_v, out_v):
        wid = lax.axis_index("s") * info.num_cores + lax.axis_index("c")
        base = wid * t_per_w
        pltpu.sync_copy(pk_hbm.at[:, pl.ds(base, t_per_w)], pk_v)

        def jbody(j, carry):
            def tbody(tg, c2):
                v = pk_v[j, pl.ds(tg * 16, 16)]
                rank16 = (v >> 26) & 63
                payload = v & 0x3FFFFFF
                addr = (lax.iota(jnp.int32, 16) + tg * 16) * _NE + rank16
                plsc.store_scatter(out_v, [addr], payload)
                return c2

            return lax.fori_loop(0, t_per_w // 16, tbody, carry, unroll=4)

        lax.fori_loop(0, _NE, jbody, 0)
        pltpu.sync_copy(out_v, out_hbm.at[pl.ds(base * _NE, t_per_w * _NE)])

    return k(pk).reshape(n, _NE)


@jax.jit
def kernel(hidden_states, weight, e_score_correction_bias):
    hs = hidden_states.reshape(-1, _HID).astype(jnp.float32)
    w = weight.astype(jnp.float32)
    bcol = e_score_correction_bias.reshape(_NE, 1).astype(jnp.float32)
    n = hs.shape[0]
    outs = []
    for h in range(2):
        pk = _tc_run(hs[h * (n // 2):(h + 1) * (n // 2)], w, bcol, tb=512)
        outs.append(_sc_scatter(pk))
    out = jnp.concatenate(outs, axis=0)
    idx = out >> 20
    wt = (out & 0xFFFFF).astype(jnp.float32) * (1.0 / _WFIX)
    return idx, wt


# speedup vs baseline: 1.0049x; 1.0049x over previous
import functools

import jax
import jax.numpy as jnp
from jax import lax
from jax.experimental import pallas as pl
from jax.experimental.pallas import tpu as pltpu
from jax.experimental.pallas import tpu_sc as plsc

_HID = 4096
_NE = 64
_NG = 8
_GS = _NE // _NG
_TKG = 4
_SCALE = 2.5
_NEG = -3.0e38
_CW = 128
_WFIX = 262144.0


def _route_chunk(scores, bias, tb):
    sfc = scores + bias

    grows = []
    for g in range(_NG):
        grp = sfc[g * _GS:(g + 1) * _GS, :]
        m1 = jnp.max(grp, axis=0, keepdims=True)
        is_m1 = grp == m1
        nmax = jnp.sum(jnp.where(is_m1, 1.0, 0.0), axis=0, keepdims=True)
        m2 = jnp.max(jnp.where(is_m1, _NEG, grp), axis=0, keepdims=True)
        m2 = jnp.where(nmax > 1.0, m1, m2)
        grows.append(m1 + m2)
    gscores = jnp.concatenate(grows, axis=0)

    giota = jax.lax.broadcasted_iota(jnp.int32, (_NG, tb), 0)
    grank = jnp.zeros((_NG, tb), jnp.float32)
    for g in range(_NG):
        vg = gscores[g:g + 1, :]
        cond = (vg > gscores) | ((vg == gscores) & (giota > g))
        grank = grank + jnp.where(cond, 1.0, 0.0)
    keep = jnp.where(grank < float(_TKG), 1.0, 0.0)
    keep_full = jnp.concatenate(
        [jnp.broadcast_to(keep[g:g + 1, :], (_GS, tb)) for g in range(_NG)],
        axis=0,
    )
    masked = jnp.where(keep_full > 0.5, sfc, 0.0)

    biota = jax.lax.broadcasted_iota(jnp.int32, (_GS, tb), 0)
    mblk = [masked[b * 8:(b + 1) * 8, :] for b in range(8)]
    rblk = [jnp.zeros((8, tb), jnp.float32) for _ in range(8)]
    for j in range(_NE):
        vj = masked[j:j + 1, :]
        jb = j // 8
        for b in range(8):
            if b < jb:
                cond = vj > mblk[b]
            elif b > jb:
                cond = vj >= mblk[b]
            else:
                cond = (vj > mblk[b]) | ((vj == mblk[b]) & (biota > (j - 8 * jb)))
            rblk[b] = rblk[b] + jnp.where(cond, 1.0, 0.0)
    rank = jnp.concatenate(rblk, axis=0)

    denom = jnp.sum(scores, axis=0, keepdims=True) + 1e-20
    wfix = (scores * (_SCALE * _WFIX) / denom).astype(jnp.int32)
    eiota = jax.lax.broadcasted_iota(jnp.int32, (_NE, tb), 0)
    pk = (rank.astype(jnp.int32) << 26) | (eiota << 20) | wfix
    return pk


def _tc_kernel(hs_ref, w_ref, b_ref, pk_ref):
    w = w_ref[...]
    bias = b_ref[...]
    tb = hs_ref.shape[0]
    logits = jax.lax.dot_general(
        w, hs_ref[...], (((1,), (1,)), ((), ())),
        preferred_element_type=jnp.float32,
    )
    for c in range(tb // _CW):
        lo, hi = c * _CW, (c + 1) * _CW
        scores = jax.nn.sigmoid(logits[:, lo:hi])
        pk_ref[:, lo:hi] = _route_chunk(scores, bias, _CW)


def _tc_run(hs, w, b2d, tb, interpret=False):
    n = hs.shape[0]
    return pl.pallas_call(
        _tc_kernel,
        grid=(n // tb,),
        in_specs=[
            pl.BlockSpec((tb, _HID), lambda i: (i, 0)),
            pl.BlockSpec((_NE, _HID), lambda i: (0, 0)),
            pl.BlockSpec((_NE, 1), lambda i: (0, 0)),
        ],
        out_specs=pl.BlockSpec((_NE, tb), lambda i: (0, i)),
        out_shape=jax.ShapeDtypeStruct((_NE, n), jnp.int32),
        interpret=interpret,
    )(hs, w, b2d)


def _sc_scatter(pk):
    n = pk.shape[1]
    info = plsc.get_sparse_core_info()
    nw = info.num_cores * info.num_subcores
    t_per_w = n // nw
    mesh = plsc.VectorSubcoreMesh(core_axis_name="c", subcore_axis_name="s")

    @functools.partial(
        pl.kernel, mesh=mesh,
        out_type=jax.ShapeDtypeStruct((n * _NE,), jnp.int32),
        scratch_types=[
            pltpu.VMEM((_NE, t_per_w), jnp.int32),
            pltpu.VMEM((t_per_w * _NE,), jnp.int32),
        ],
        compiler_params=pltpu.CompilerParams(needs_layout_passes=False),
    )
    def k(pk_hbm, out_hbm, pk_v, out_v):
        wid = lax.axis_index("s") * info.num_cores + lax.axis_index("c")
        base = wid * t_per_w
        pltpu.sync_copy(pk_hbm.at[:, pl.ds(base, t_per_w)], pk_v)

        def body(step, carry):
            j = step // (t_per_w // 16)
            tg = step % (t_per_w // 16)
            v = pk_v[j, pl.ds(tg * 16, 16)]
            rank16 = (v >> 26) & 63
            payload = v & 0x3FFFFFF
            addr = (lax.iota(jnp.int32, 16) + tg * 16) * _NE + rank16
            plsc.store_scatter(out_v, [addr], payload)
            return carry

        lax.fori_loop(0, _NE * (t_per_w // 16), body, 0)
        pltpu.sync_copy(out_v, out_hbm.at[pl.ds(base * _NE, t_per_w * _NE)])

    return k(pk).reshape(n, _NE)


@jax.jit
def kernel(hidden_states, weight, e_score_correction_bias):
    hs = hidden_states.reshape(-1, _HID).astype(jnp.float32)
    w = weight.astype(jnp.float32)
    bcol = e_score_correction_bias.reshape(_NE, 1).astype(jnp.float32)
    n = hs.shape[0]
    outs = []
    for h in range(2):
        pk = _tc_run(hs[h * (n // 2):(h + 1) * (n // 2)], w, bcol, tb=512)
        outs.append(_sc_scatter(pk))
    out = jnp.concatenate(outs, axis=0)
    idx = out >> 20
    wt = (out & 0xFFFFF).astype(jnp.float32) * (1.0 / _WFIX)
    return idx, wt


# speedup vs baseline: 2.7388x; 2.7256x over previous
import jax
import jax.numpy as jnp
from jax.experimental import pallas as pl

_HID = 4096
_NE = 64
_NG = 8
_GS = _NE // _NG
_TKG = 4
_SCALE = 2.5
_NEG = -3.0e38


_CW = 128


def _route_chunk(scores, bias, tb):
    sfc = scores + bias

    grows = []
    for g in range(_NG):
        grp = sfc[g * _GS:(g + 1) * _GS, :]
        m1 = jnp.max(grp, axis=0, keepdims=True)
        is_m1 = grp == m1
        nmax = jnp.sum(jnp.where(is_m1, 1.0, 0.0), axis=0, keepdims=True)
        m2 = jnp.max(jnp.where(is_m1, _NEG, grp), axis=0, keepdims=True)
        m2 = jnp.where(nmax > 1.0, m1, m2)
        grows.append(m1 + m2)
    gscores = jnp.concatenate(grows, axis=0)

    giota = jax.lax.broadcasted_iota(jnp.int32, (_NG, tb), 0)
    grank = jnp.zeros((_NG, tb), jnp.float32)
    for g in range(_NG):
        vg = gscores[g:g + 1, :]
        cond = (vg > gscores) | ((vg == gscores) & (giota > g))
        grank = grank + jnp.where(cond, 1.0, 0.0)
    keep = jnp.where(grank < float(_TKG), 1.0, 0.0)
    keep_full = jnp.concatenate(
        [jnp.broadcast_to(keep[g:g + 1, :], (_GS, tb)) for g in range(_NG)],
        axis=0,
    )
    masked = jnp.where(keep_full > 0.5, sfc, 0.0)

    biota = jax.lax.broadcasted_iota(jnp.int32, (_GS, tb), 0)
    mblk = [masked[b * 8:(b + 1) * 8, :] for b in range(8)]
    rblk = [jnp.zeros((8, tb), jnp.float32) for _ in range(8)]
    for j in range(_NE):
        vj = masked[j:j + 1, :]
        jb = j // 8
        for b in range(8):
            if b < jb:
                cond = vj > mblk[b]
            elif b > jb:
                cond = vj >= mblk[b]
            else:
                cond = (vj > mblk[b]) | ((vj == mblk[b]) & (biota > (j - 8 * jb)))
            rblk[b] = rblk[b] + jnp.where(cond, 1.0, 0.0)

    pblk = [biota.astype(jnp.float32) + float(8 * b) for b in range(8)]
    packed = [pblk[b] * 64.0 + scores[b * 8:(b + 1) * 8, :] * 32.0
              for b in range(8)]
    ablk = [jnp.zeros((8, tb), jnp.float32) for _ in range(8)]
    for j in range(_NE):
        jb, jr = j // 8, j % 8
        rrow = rblk[jb][jr:jr + 1, :]
        prow = packed[jb][jr:jr + 1, :]
        for b in range(8):
            hit = rrow == pblk[b]
            ablk[b] = ablk[b] + jnp.where(hit, prow, 0.0)

    acc = jnp.concatenate(ablk, axis=0)
    idx = jnp.floor(acc * (1.0 / 64.0))
    wsel = (acc - idx * 64.0) * (1.0 / 32.0)
    denom = jnp.sum(scores, axis=0, keepdims=True) + 1e-20
    return idx, wsel, denom


def _router_kernel(hs_ref, w_ref, b_ref, idx_ref, wt_ref):
    w = w_ref[...]
    bias = b_ref[...]
    tb = hs_ref.shape[0]
    for h in range(2):
        hlo = h * (tb // 2)
        logits = jax.lax.dot_general(
            w, hs_ref[hlo:hlo + tb // 2, :], (((1,), (1,)), ((), ())),
            preferred_element_type=jnp.float32,
        )
        for c in range(tb // 2 // _CW):
            lo, hi = c * _CW, (c + 1) * _CW
            scores = jax.nn.sigmoid(logits[:, lo:hi])
            idx, wsel, denom = _route_chunk(scores, bias, _CW)
            idx_ref[:, hlo + lo:hlo + hi] = idx.astype(jnp.int32)
            wt_ref[:, hlo + lo:hlo + hi] = (wsel / denom) * _SCALE


def _run(hs, w, b2d, tb, interpret=False):
    n = hs.shape[0]
    return pl.pallas_call(
        _router_kernel,
        grid=(n // tb,),
        in_specs=[
            pl.BlockSpec((tb, _HID), lambda i: (i, 0)),
            pl.BlockSpec((_NE, _HID), lambda i: (0, 0)),
            pl.BlockSpec((_NE, 1), lambda i: (0, 0)),
        ],
        out_specs=[
            pl.BlockSpec((_NE, tb), lambda i: (0, i)),
            pl.BlockSpec((_NE, tb), lambda i: (0, i)),
        ],
        out_shape=[
            jax.ShapeDtypeStruct((_NE, n), jnp.int32),
            jax.ShapeDtypeStruct((_NE, n), jnp.float32),
        ],
        interpret=interpret,
    )(hs, w, b2d)


@jax.jit
def kernel(hidden_states, weight, e_score_correction_bias):
    hs = hidden_states.reshape(-1, _HID).astype(jnp.float32)
    w = weight.astype(jnp.float32)
    bcol = e_score_correction_bias.reshape(_NE, 1).astype(jnp.float32)
    idx_t, wt_t = _run(hs, w, bcol, tb=1024)
    return idx_t.T, wt_t.T


# speedup vs baseline: 2.8426x; 1.0379x over previous
import jax
import jax.numpy as jnp
from jax.experimental import pallas as pl
from jax.experimental.pallas import tpu as pltpu

_HID = 4096
_NE = 64
_NG = 8
_GS = _NE // _NG
_TKG = 4
_SCALE = 2.5
_NEG = -3.0e38


_CW = 128


def _route_chunk(scores, bias, tb):
    sfc = scores + bias

    grows = []
    for g in range(_NG):
        grp = sfc[g * _GS:(g + 1) * _GS, :]
        m1 = jnp.max(grp, axis=0, keepdims=True)
        is_m1 = grp == m1
        nmax = jnp.sum(jnp.where(is_m1, 1.0, 0.0), axis=0, keepdims=True)
        m2 = jnp.max(jnp.where(is_m1, _NEG, grp), axis=0, keepdims=True)
        m2 = jnp.where(nmax > 1.0, m1, m2)
        grows.append(m1 + m2)
    gscores = jnp.concatenate(grows, axis=0)

    giota = jax.lax.broadcasted_iota(jnp.int32, (_NG, tb), 0)
    grank = jnp.zeros((_NG, tb), jnp.float32)
    for g in range(_NG):
        vg = gscores[g:g + 1, :]
        cond = (vg > gscores) | ((vg == gscores) & (giota > g))
        grank = grank + jnp.where(cond, 1.0, 0.0)
    keep = jnp.where(grank < float(_TKG), 1.0, 0.0)
    keep_full = jnp.concatenate(
        [jnp.broadcast_to(keep[g:g + 1, :], (_GS, tb)) for g in range(_NG)],
        axis=0,
    )
    masked = jnp.where(keep_full > 0.5, sfc, 0.0)

    biota = jax.lax.broadcasted_iota(jnp.int32, (_GS, tb), 0)
    mblk = [masked[b * 8:(b + 1) * 8, :] for b in range(8)]
    rblk = [jnp.zeros((8, tb), jnp.float32) for _ in range(8)]
    for j in range(_NE):
        vj = masked[j:j + 1, :]
        jb = j // 8
        for b in range(8):
            if b < jb:
                cond = vj > mblk[b]
            elif b > jb:
                cond = vj >= mblk[b]
            else:
                cond = (vj > mblk[b]) | ((vj == mblk[b]) & (biota > (j - 8 * jb)))
            rblk[b] = rblk[b] + jnp.where(cond, 1.0, 0.0)

    pblk = [biota.astype(jnp.float32) + float(8 * b) for b in range(8)]
    packed = [pblk[b] * 64.0 + scores[b * 8:(b + 1) * 8, :] * 32.0
              for b in range(8)]
    ablk = [jnp.zeros((8, tb), jnp.float32) for _ in range(8)]
    for j in range(_NE):
        jb, jr = j // 8, j % 8
        rrow = rblk[jb][jr:jr + 1, :]
        prow = packed[jb][jr:jr + 1, :]
        for b in range(8):
            hit = rrow == pblk[b]
            ablk[b] = ablk[b] + jnp.where(hit, prow, 0.0)

    acc = jnp.concatenate(ablk, axis=0)
    idx = jnp.floor(acc * (1.0 / 64.0))
    wsel = (acc - idx * 64.0) * (1.0 / 32.0)
    denom = jnp.sum(scores, axis=0, keepdims=True) + 1e-20
    return idx, wsel, denom


def _router_kernel(hs_ref, w_ref, b_ref, idx_ref, wt_ref, lg_ref):
    w = w_ref[...]
    bias = b_ref[...]
    tb = lg_ref.shape[1]
    for c in range(tb // _CW):
        lo, hi = c * _CW, (c + 1) * _CW
        scores = jax.nn.sigmoid(lg_ref[:, lo:hi])
        idx, wsel, denom = _route_chunk(scores, bias, _CW)
        idx_ref[:, lo:hi] = idx.astype(jnp.int32)
        wt_ref[:, lo:hi] = (wsel / denom) * _SCALE
    lg_ref[...] = jax.lax.dot_general(
        w, hs_ref[...], (((1,), (1,)), ((), ())),
        preferred_element_type=jnp.float32,
    )


def _run(hs, w, b2d, tb, interpret=False):
    n = hs.shape[0]
    nblk = n // tb
    return pl.pallas_call(
        _router_kernel,
        grid=(nblk + 1,),
        in_specs=[
            pl.BlockSpec((tb, _HID), lambda i: (jnp.minimum(i, nblk - 1), 0)),
            pl.BlockSpec((_NE, _HID), lambda i: (0, 0)),
            pl.BlockSpec((_NE, 1), lambda i: (0, 0)),
        ],
        out_specs=[
            pl.BlockSpec((_NE, tb), lambda i: (0, jnp.maximum(i - 1, 0))),
            pl.BlockSpec((_NE, tb), lambda i: (0, jnp.maximum(i - 1, 0))),
        ],
        out_shape=[
            jax.ShapeDtypeStruct((_NE, n), jnp.int32),
            jax.ShapeDtypeStruct((_NE, n), jnp.float32),
        ],
        scratch_shapes=[pltpu.VMEM((_NE, tb), jnp.float32)],
        interpret=interpret,
    )(hs, w, b2d)


@jax.jit
def kernel(hidden_states, weight, e_score_correction_bias):
    hs = hidden_states.reshape(-1, _HID).astype(jnp.float32)
    w = weight.astype(jnp.float32)
    bcol = e_score_correction_bias.reshape(_NE, 1).astype(jnp.float32)
    idx_t, wt_t = _run(hs, w, bcol, tb=512)
    return idx_t.T, wt_t.T


# speedup vs baseline: 2.9445x; 1.0358x over previous
import jax
import jax.numpy as jnp
from jax.experimental import pallas as pl
from jax.experimental.pallas import tpu as pltpu

_HID = 4096
_NE = 64
_NG = 8
_GS = _NE // _NG
_TKG = 4
_SCALE = 2.5
_NEG = -3.0e38


_CW = 128


def _route_chunk(scores, bias, tb):
    sfc = scores + bias

    grows = []
    for g in range(_NG):
        grp = sfc[g * _GS:(g + 1) * _GS, :]
        m1 = jnp.max(grp, axis=0, keepdims=True)
        is_m1 = grp == m1
        nmax = jnp.sum(jnp.where(is_m1, 1.0, 0.0), axis=0, keepdims=True)
        m2 = jnp.max(jnp.where(is_m1, _NEG, grp), axis=0, keepdims=True)
        m2 = jnp.where(nmax > 1.0, m1, m2)
        grows.append(m1 + m2)
    gscores = jnp.concatenate(grows, axis=0)

    giota = jax.lax.broadcasted_iota(jnp.int32, (_NG, tb), 0)
    grank = jnp.zeros((_NG, tb), jnp.float32)
    for g in range(_NG):
        vg = gscores[g:g + 1, :]
        cond = (vg > gscores) | ((vg == gscores) & (giota > g))
        grank = grank + jnp.where(cond, 1.0, 0.0)
    keep = jnp.where(grank < float(_TKG), 1.0, 0.0)
    keep_full = jnp.concatenate(
        [jnp.broadcast_to(keep[g:g + 1, :], (_GS, tb)) for g in range(_NG)],
        axis=0,
    )
    masked = jnp.where(keep_full > 0.5, sfc, 0.0)

    biota = jax.lax.broadcasted_iota(jnp.int32, (_GS, tb), 0)
    mblk = [masked[b * 8:(b + 1) * 8, :] for b in range(8)]
    rblk = [jnp.zeros((8, tb), jnp.float32) for _ in range(8)]
    for j in range(_NE):
        vj = masked[j:j + 1, :]
        jb = j // 8
        for b in range(8):
            if b < jb:
                cond = vj > mblk[b]
            elif b > jb:
                cond = vj >= mblk[b]
            else:
                cond = (vj > mblk[b]) | ((vj == mblk[b]) & (biota > (j - 8 * jb)))
            rblk[b] = rblk[b] + jnp.where(cond, 1.0, 0.0)

    pblk = [biota.astype(jnp.float32) + float(8 * b) for b in range(8)]
    packed = [pblk[b] * 64.0 + scores[b * 8:(b + 1) * 8, :] * 32.0
              for b in range(8)]
    ablk = [jnp.zeros((8, tb), jnp.float32) for _ in range(8)]
    for j in range(_NE):
        jb, jr = j // 8, j % 8
        rrow = rblk[jb][jr:jr + 1, :]
        prow = packed[jb][jr:jr + 1, :]
        for b in range(8):
            hit = rrow == pblk[b]
            ablk[b] = jnp.where(hit, prow, ablk[b])

    acc = jnp.concatenate(ablk, axis=0)
    idx = jnp.floor(acc * (1.0 / 64.0))
    wsel = (acc - idx * 64.0) * (1.0 / 32.0)
    denom = jnp.sum(scores, axis=0, keepdims=True) + 1e-20
    return idx, wsel, denom


def _router_kernel(hs_ref, w_ref, b_ref, idx_ref, wt_ref, lg_ref):
    w = w_ref[...]
    bias = b_ref[...]
    tb = lg_ref.shape[1]
    for c in range(tb // _CW):
        lo, hi = c * _CW, (c + 1) * _CW
        scores = jax.nn.sigmoid(lg_ref[:, lo:hi])
        idx, wsel, denom = _route_chunk(scores, bias, _CW)
        idx_ref[:, lo:hi] = idx.astype(jnp.int32)
        wt_ref[:, lo:hi] = (wsel / denom) * _SCALE
    lg_ref[...] = jax.lax.dot_general(
        w, hs_ref[...], (((1,), (1,)), ((), ())),
        preferred_element_type=jnp.float32,
    )


def _run(hs, w, b2d, tb, interpret=False):
    n = hs.shape[0]
    nblk = n // tb
    return pl.pallas_call(
        _router_kernel,
        grid=(nblk + 1,),
        in_specs=[
            pl.BlockSpec((tb, _HID), lambda i: (jnp.minimum(i, nblk - 1), 0)),
            pl.BlockSpec((_NE, _HID), lambda i: (0, 0)),
            pl.BlockSpec((_NE, 1), lambda i: (0, 0)),
        ],
        out_specs=[
            pl.BlockSpec((_NE, tb), lambda i: (0, jnp.maximum(i - 1, 0))),
            pl.BlockSpec((_NE, tb), lambda i: (0, jnp.maximum(i - 1, 0))),
        ],
        out_shape=[
            jax.ShapeDtypeStruct((_NE, n), jnp.int32),
            jax.ShapeDtypeStruct((_NE, n), jnp.float32),
        ],
        scratch_shapes=[pltpu.VMEM((_NE, tb), jnp.float32)],
        interpret=interpret,
    )(hs, w, b2d)


@jax.jit
def kernel(hidden_states, weight, e_score_correction_bias):
    hs = hidden_states.reshape(-1, _HID).astype(jnp.float32)
    w = weight.astype(jnp.float32)
    bcol = e_score_correction_bias.reshape(_NE, 1).astype(jnp.float32)
    idx_t, wt_t = _run(hs, w, bcol, tb=512)
    return idx_t.T, wt_t.T
